# baseline (speedup 1.0000x reference)
import functools

import jax
import jax.numpy as jnp
from jax import lax
from jax.experimental import pallas as pl
from jax.experimental.pallas import tpu as pltpu
from jax.experimental.pallas import tpu_sc as plsc

NC = 2
NS = 16
NW = NC * NS
CH = 128
DEG_K = 8
EPS = 1e-5


def _degree_call(dst2, zeros_nd, ones2d, np_, d, n_chunks):
    @functools.partial(
        pl.kernel,
        out_type=jax.ShapeDtypeStruct((NC, np_, d), jnp.float32),
        mesh=plsc.VectorSubcoreMesh(core_axis_name="c", subcore_axis_name="s",
                                    num_cores=NC, num_subcores=NS),
        scratch_types=[
            pltpu.VMEM((n_chunks, CH), jnp.int32),
            pltpu.VMEM((CH, d), jnp.float32),
            pltpu.VMEM_SHARED((np_, d), jnp.float32),
            pltpu.SemaphoreType.DMA,
        ],
    )
    def deg_kernel(dst_hbm, z_hbm, ones_hbm, out_hbm, didx, ones_v, dacc, sem):
        cid = lax.axis_index("c")
        sid = lax.axis_index("s")
        wid = sid * NC + cid
        rpt = np_ // NS
        rbase = sid * rpt
        pltpu.sync_copy(z_hbm.at[pl.ds(rbase, rpt)], dacc.at[pl.ds(rbase, rpt)])
        pltpu.sync_copy(ones_hbm, ones_v)
        pltpu.sync_copy(dst_hbm.at[pl.ds(wid * n_chunks, n_chunks)], didx)
        plsc.subcore_barrier()

        @pl.loop(0, n_chunks, step=DEG_K)
        def _(j):
            descs = [
                pltpu.async_copy(ones_v, dacc.at[didx.at[j + k]], sem, add=True)
                for k in range(DEG_K)
            ]
            for desc in descs:
                desc.wait()

        plsc.subcore_barrier()
        pltpu.sync_copy(dacc.at[pl.ds(rbase, rpt)],
                        out_hbm.at[cid].at[pl.ds(rbase, rpt)])

    return deg_kernel(dst2, zeros_nd, ones2d)


def _aggregate_call(hs, src2, dst2, np_, d, n_chunks):
    @functools.partial(
        pl.kernel,
        out_type=jax.ShapeDtypeStruct((NC, np_, d), jnp.float32),
        mesh=plsc.VectorSubcoreMesh(core_axis_name="c", subcore_axis_name="s",
                                    num_cores=NC, num_subcores=NS),
        scratch_types=[
            pltpu.VMEM((n_chunks // 2, CH), jnp.int32),
            pltpu.VMEM((n_chunks // 2, CH), jnp.int32),
            pltpu.VMEM((CH, d), jnp.float32),
            pltpu.VMEM((CH, d), jnp.float32),
            pltpu.VMEM_SHARED((np_, d), jnp.float32),
            pltpu.SemaphoreType.DMA,
            pltpu.SemaphoreType.DMA,
        ],
    )
    def agg_kernel(hs_hbm, src_hbm, dst_hbm, out_hbm,
                   sidx, didx, rows_a, rows_b, acc, sem_a, sem_b):
        cid = lax.axis_index("c")
        sid = lax.axis_index("s")
        wid = sid * NC + cid
        rpt = np_ // NS
        rbase = sid * rpt
        hc = n_chunks // 2
        pltpu.sync_copy(hs_hbm.at[pl.ds(rbase, rpt)], acc.at[pl.ds(rbase, rpt)])
        plsc.subcore_barrier()

        for half in range(2):
            cbase = wid * n_chunks + half * hc
            pltpu.sync_copy(src_hbm.at[pl.ds(cbase, hc)], sidx)
            pltpu.sync_copy(dst_hbm.at[pl.ds(cbase, hc)], didx)
            pltpu.async_copy(hs_hbm.at[sidx.at[0]], rows_a, sem_a).wait()

            @pl.loop(0, hc, step=2)
            def _(j):
                gb = pltpu.async_copy(hs_hbm.at[sidx.at[j + 1]], rows_b, sem_b)
                pltpu.sync_copy(rows_a, acc.at[didx.at[j]], add=True)
                gb.wait()

                @pl.when(j + 2 < hc)
                def _():
                    pltpu.async_copy(hs_hbm.at[sidx.at[j + 2]], rows_a, sem_a)

                pltpu.sync_copy(rows_b, acc.at[didx.at[j + 1]], add=True)

                @pl.when(j + 2 < hc)
                def _():
                    pltpu.make_async_copy(hs_hbm.at[sidx.at[0]], rows_a,
                                          sem_a).wait()

        plsc.subcore_barrier()
        pltpu.sync_copy(acc.at[pl.ds(rbase, rpt)],
                        out_hbm.at[cid].at[pl.ds(rbase, rpt)])

    return agg_kernel(hs, src2, dst2)


def _matmul_call(x, w):
    def mm_kernel(x_ref, w_ref, o_ref):
        o_ref[...] = jnp.dot(x_ref[...], w_ref[...],
                             preferred_element_type=jnp.float32)

    return pl.pallas_call(
        mm_kernel,
        out_shape=jax.ShapeDtypeStruct((x.shape[0], w.shape[1]), jnp.float32),
    )(x, w)


def _scale_first_call(h, degs, np_, d):
    def k(h_ref, dg_ref, hs_ref, dinv_ref):
        deg = dg_ref[0, :, 0:1] + dg_ref[1, :, 0:1] + 1.0
        dinv = lax.rsqrt(deg)
        dinv_ref[...] = dinv
        hs_ref[...] = h_ref[...] * dinv

    return pl.pallas_call(
        k,
        out_shape=[
            jax.ShapeDtypeStruct((np_, d), jnp.float32),
            jax.ShapeDtypeStruct((np_, 1), jnp.float32),
        ],
    )(h, degs)


def _mid_layer_call(acc, hs_prev, dinv, b, g, be, w_next, n, np_, d):
    def k(acc_ref, hsp_ref, dinv_ref, b_ref, g_ref, be_ref, w_ref, o_ref):
        dinv_v = dinv_ref[...]
        agg = acc_ref[0] + acc_ref[1] - hsp_ref[...]
        y = agg * dinv_v + b_ref[...]
        yr = y[:n, :]
        mu = jnp.mean(yr, axis=0, keepdims=True)
        cr = yr - mu
        var = jnp.mean(cr * cr, axis=0, keepdims=True)
        yn = g_ref[...] * ((y - mu) * lax.rsqrt(var + EPS)) + be_ref[...]
        r = jnp.maximum(yn, 0.0)
        o_ref[...] = jnp.dot(r, w_ref[...],
                             preferred_element_type=jnp.float32) * dinv_v

    return pl.pallas_call(
        k,
        out_shape=jax.ShapeDtypeStruct((np_, d), jnp.float32),
    )(acc, hs_prev, dinv, b, g, be, w_next)


def _final_layer_call(acc, hs_prev, dinv, b, n, d):
    def k(acc_ref, hsp_ref, dinv_ref, b_ref, o_ref):
        agg = acc_ref[0, :n, :] + acc_ref[1, :n, :] - hsp_ref[:n, :]
        o_ref[...] = agg * dinv_ref[:n, :] + b_ref[...]

    return pl.pallas_call(
        k,
        out_shape=jax.ShapeDtypeStruct((n, d), jnp.float32),
    )(acc, hs_prev, dinv, b)


def kernel(x, edge_index, W1, b1, g1, be1, W2, b2, g2, be2, W3, b3):
    n, d = x.shape
    e = edge_index.shape[1]

    np_ = -(-n // 128) * 128
    epw = -(-e // (NW * CH * 8)) * CH * 8
    ep = epw * NW
    n_chunks = epw // CH
    pad = ep - e
    src = edge_index[0]
    dst = edge_index[1]
    src2 = jnp.concatenate([src, jnp.full((pad,), n, jnp.int32)]).reshape(-1, CH)
    dst2 = jnp.concatenate([dst, jnp.full((pad,), n, jnp.int32)]).reshape(-1, CH)
    zeros_nd = jnp.zeros((np_, d), jnp.float32)
    ones2d = jnp.ones((CH, d), jnp.float32)
    x_p = jnp.concatenate([x, jnp.zeros((np_ - n, d), jnp.float32)])

    degs = _degree_call(dst2, zeros_nd, ones2d, np_, d, n_chunks)
    h1 = _matmul_call(x_p, W1)
    hs1, dinv = _scale_first_call(h1, degs, np_, d)

    b1r, g1r, be1r = b1.reshape(1, d), g1.reshape(1, d), be1.reshape(1, d)
    b2r, g2r, be2r = b2.reshape(1, d), g2.reshape(1, d), be2.reshape(1, d)
    b3r = b3.reshape(1, d)

    acc1 = _aggregate_call(hs1, src2, dst2, np_, d, n_chunks)
    hs2 = _mid_layer_call(acc1, hs1, dinv, b1r, g1r, be1r, W2, n, np_, d)
    acc2 = _aggregate_call(hs2, src2, dst2, np_, d, n_chunks)
    hs3 = _mid_layer_call(acc2, hs2, dinv, b2r, g2r, be2r, W3, n, np_, d)
    acc3 = _aggregate_call(hs3, src2, dst2, np_, d, n_chunks)
    return _final_layer_call(acc3, hs3, dinv, b3r, n, d)


# speedup vs baseline: 1.1016x; 1.1016x over previous
import functools

import jax
import jax.numpy as jnp
from jax import lax
from jax.experimental import pallas as pl
from jax.experimental.pallas import tpu as pltpu
from jax.experimental.pallas import tpu_sc as plsc

NC = 2
NS = 16
NW = NC * NS
CH = 128
DEG_K = 8
BSZ = 32
CORE0_NUM, CORE_DEN = 4, 5
EPS = 1e-5


def _degree_call(dst2, zeros_nd, ones2d, np_, d, n_chunks):
    @functools.partial(
        pl.kernel,
        out_type=jax.ShapeDtypeStruct((NC, np_, d), jnp.float32),
        mesh=plsc.VectorSubcoreMesh(core_axis_name="c", subcore_axis_name="s",
                                    num_cores=NC, num_subcores=NS),
        scratch_types=[
            pltpu.VMEM((n_chunks, CH), jnp.int32),
            pltpu.VMEM((CH, d), jnp.float32),
            pltpu.VMEM_SHARED((np_, d), jnp.float32),
            pltpu.SemaphoreType.DMA,
        ],
    )
    def deg_kernel(dst_hbm, z_hbm, ones_hbm, out_hbm, didx, ones_v, dacc, sem):
        cid = lax.axis_index("c")
        sid = lax.axis_index("s")
        wid = sid * NC + cid
        rpt = np_ // NS
        rbase = sid * rpt
        pltpu.sync_copy(z_hbm.at[pl.ds(rbase, rpt)], dacc.at[pl.ds(rbase, rpt)])
        pltpu.sync_copy(ones_hbm, ones_v)
        pltpu.sync_copy(dst_hbm.at[pl.ds(wid * n_chunks, n_chunks)], didx)
        plsc.subcore_barrier()

        @pl.loop(0, n_chunks, step=DEG_K)
        def _(j):
            descs = [
                pltpu.async_copy(ones_v, dacc.at[didx.at[j + k]], sem, add=True)
                for k in range(DEG_K)
            ]
            for desc in descs:
                desc.wait()

        plsc.subcore_barrier()
        pltpu.sync_copy(dacc.at[pl.ds(rbase, rpt)],
                        out_hbm.at[cid].at[pl.ds(rbase, rpt)])

    return deg_kernel(dst2, zeros_nd, ones2d)


def _aggregate_call(hs, src2, dst2, np_, d, n_chunks):
    @functools.partial(
        pl.kernel,
        out_type=jax.ShapeDtypeStruct((NC, np_, d), jnp.float32),
        mesh=plsc.VectorSubcoreMesh(core_axis_name="c", subcore_axis_name="s",
                                    num_cores=NC, num_subcores=NS),
        scratch_types=[
            pltpu.VMEM((BSZ, CH), jnp.int32),
            pltpu.VMEM((BSZ, CH), jnp.int32),
            pltpu.VMEM((CH, d), jnp.float32),
            pltpu.VMEM((CH, d), jnp.float32),
            pltpu.VMEM_SHARED((np_, d), jnp.float32),
            pltpu.SemaphoreType.DMA,
            pltpu.SemaphoreType.DMA,
        ],
    )
    def agg_kernel(hs_hbm, src_hbm, dst_hbm, out_hbm,
                   sidx, didx, rows_a, rows_b, acc, sem_a, sem_b):
        cid = lax.axis_index("c")
        sid = lax.axis_index("s")
        rpt = np_ // NS
        rbase = sid * rpt
        pltpu.sync_copy(hs_hbm.at[pl.ds(rbase, rpt)], acc.at[pl.ds(rbase, rpt)])
        plsc.subcore_barrier()

        def process(base_chunk, nblocks):
            for blk in range(nblocks):
                cb = base_chunk + blk * BSZ
                pltpu.sync_copy(src_hbm.at[pl.ds(cb, BSZ)], sidx)
                pltpu.sync_copy(dst_hbm.at[pl.ds(cb, BSZ)], didx)
                pltpu.async_copy(hs_hbm.at[sidx.at[0]], rows_a, sem_a).wait()

                @pl.loop(0, BSZ, step=2)
                def _(j):
                    gb = pltpu.async_copy(hs_hbm.at[sidx.at[j + 1]], rows_b,
                                          sem_b)
                    pltpu.sync_copy(rows_a, acc.at[didx.at[j]], add=True)
                    gb.wait()

                    @pl.when(j + 2 < BSZ)
                    def _():
                        pltpu.async_copy(hs_hbm.at[sidx.at[j + 2]], rows_a,
                                         sem_a)

                    pltpu.sync_copy(rows_b, acc.at[didx.at[j + 1]], add=True)

                    @pl.when(j + 2 < BSZ)
                    def _():
                        pltpu.make_async_copy(hs_hbm.at[sidx.at[0]], rows_a,
                                              sem_a).wait()

        k0 = (n_chunks * 2 * CORE0_NUM) // CORE_DEN
        k1 = n_chunks * 2 - k0

        @pl.when(cid == 0)
        def _():
            process(sid * k0, k0 // BSZ)

        @pl.when(cid == 1)
        def _():
            process(NS * k0 + sid * k1, k1 // BSZ)

        plsc.subcore_barrier()
        pltpu.sync_copy(acc.at[pl.ds(rbase, rpt)],
                        out_hbm.at[cid].at[pl.ds(rbase, rpt)])

    return agg_kernel(hs, src2, dst2)


def _matmul_call(x, w):
    def mm_kernel(x_ref, w_ref, o_ref):
        o_ref[...] = jnp.dot(x_ref[...], w_ref[...],
                             preferred_element_type=jnp.float32)

    return pl.pallas_call(
        mm_kernel,
        out_shape=jax.ShapeDtypeStruct((x.shape[0], w.shape[1]), jnp.float32),
    )(x, w)


def _scale_first_call(h, degs, np_, d):
    def k(h_ref, dg_ref, hs_ref, dinv_ref):
        deg = dg_ref[0, :, 0:1] + dg_ref[1, :, 0:1] + 1.0
        dinv = lax.rsqrt(deg)
        dinv_ref[...] = dinv
        hs_ref[...] = h_ref[...] * dinv

    return pl.pallas_call(
        k,
        out_shape=[
            jax.ShapeDtypeStruct((np_, d), jnp.float32),
            jax.ShapeDtypeStruct((np_, 1), jnp.float32),
        ],
    )(h, degs)


def _mid_layer_call(acc, hs_prev, dinv, b, g, be, w_next, n, np_, d):
    def k(acc_ref, hsp_ref, dinv_ref, b_ref, g_ref, be_ref, w_ref, o_ref):
        dinv_v = dinv_ref[...]
        agg = acc_ref[0] + acc_ref[1] - hsp_ref[...]
        y = agg * dinv_v + b_ref[...]
        yr = y[:n, :]
        mu = jnp.mean(yr, axis=0, keepdims=True)
        cr = yr - mu
        var = jnp.mean(cr * cr, axis=0, keepdims=True)
        yn = g_ref[...] * ((y - mu) * lax.rsqrt(var + EPS)) + be_ref[...]
        r = jnp.maximum(yn, 0.0)
        o_ref[...] = jnp.dot(r, w_ref[...],
                             preferred_element_type=jnp.float32) * dinv_v

    return pl.pallas_call(
        k,
        out_shape=jax.ShapeDtypeStruct((np_, d), jnp.float32),
    )(acc, hs_prev, dinv, b, g, be, w_next)


def _final_layer_call(acc, hs_prev, dinv, b, n, d):
    def k(acc_ref, hsp_ref, dinv_ref, b_ref, o_ref):
        agg = acc_ref[0, :n, :] + acc_ref[1, :n, :] - hsp_ref[:n, :]
        o_ref[...] = agg * dinv_ref[:n, :] + b_ref[...]

    return pl.pallas_call(
        k,
        out_shape=jax.ShapeDtypeStruct((n, d), jnp.float32),
    )(acc, hs_prev, dinv, b)


def kernel(x, edge_index, W1, b1, g1, be1, W2, b2, g2, be2, W3, b3):
    n, d = x.shape
    e = edge_index.shape[1]

    np_ = -(-n // 128) * 128
    epw = -(-e // (NW * CH * 8)) * CH * 8
    ep = epw * NW
    n_chunks = epw // CH
    pad = ep - e
    src = edge_index[0]
    dst = edge_index[1]
    src2 = jnp.concatenate([src, jnp.full((pad,), n, jnp.int32)]).reshape(-1, CH)
    dst2 = jnp.concatenate([dst, jnp.full((pad,), n, jnp.int32)]).reshape(-1, CH)
    zeros_nd = jnp.zeros((np_, d), jnp.float32)
    ones2d = jnp.ones((CH, d), jnp.float32)
    x_p = jnp.concatenate([x, jnp.zeros((np_ - n, d), jnp.float32)])

    degs = _degree_call(dst2, zeros_nd, ones2d, np_, d, n_chunks)
    h1 = _matmul_call(x_p, W1)
    hs1, dinv = _scale_first_call(h1, degs, np_, d)

    b1r, g1r, be1r = b1.reshape(1, d), g1.reshape(1, d), be1.reshape(1, d)
    b2r, g2r, be2r = b2.reshape(1, d), g2.reshape(1, d), be2.reshape(1, d)
    b3r = b3.reshape(1, d)

    acc1 = _aggregate_call(hs1, src2, dst2, np_, d, n_chunks)
    hs2 = _mid_layer_call(acc1, hs1, dinv, b1r, g1r, be1r, W2, n, np_, d)
    acc2 = _aggregate_call(hs2, src2, dst2, np_, d, n_chunks)
    hs3 = _mid_layer_call(acc2, hs2, dinv, b2r, g2r, be2r, W3, n, np_, d)
    acc3 = _aggregate_call(hs3, src2, dst2, np_, d, n_chunks)
    return _final_layer_call(acc3, hs3, dinv, b3r, n, d)


# speedup vs baseline: 2.7108x; 2.4607x over previous
import functools

import jax
import jax.numpy as jnp
from jax import lax
from jax.experimental import pallas as pl
from jax.experimental.pallas import tpu as pltpu
from jax.experimental.pallas import tpu_sc as plsc

NC = 2
NS = 16
NW = NC * NS
CH = 128
DEG_K = 8
BSZ = 16
CORE0_NUM, CORE_DEN = 1, 2
EPS = 1e-5


def _degree_call(dst2, zeros_nd, ones2d, np_, d, n_chunks):
    @functools.partial(
        pl.kernel,
        out_type=jax.ShapeDtypeStruct((NC, np_, d), jnp.float32),
        mesh=plsc.VectorSubcoreMesh(core_axis_name="c", subcore_axis_name="s",
                                    num_cores=NC, num_subcores=NS),
        scratch_types=[
            pltpu.VMEM((n_chunks, CH), jnp.int32),
            pltpu.VMEM((CH, d), jnp.float32),
            pltpu.VMEM_SHARED((np_, d), jnp.float32),
            pltpu.SemaphoreType.DMA,
        ],
    )
    def deg_kernel(dst_hbm, z_hbm, ones_hbm, out_hbm, didx, ones_v, dacc, sem):
        cid = lax.axis_index("c")
        sid = lax.axis_index("s")
        wid = sid * NC + cid
        rpt = np_ // NS
        rbase = sid * rpt
        pltpu.sync_copy(z_hbm.at[pl.ds(rbase, rpt)], dacc.at[pl.ds(rbase, rpt)])
        pltpu.sync_copy(ones_hbm, ones_v)
        pltpu.sync_copy(dst_hbm.at[pl.ds(wid * n_chunks, n_chunks)], didx)
        plsc.subcore_barrier()

        @pl.loop(0, n_chunks, step=DEG_K)
        def _(j):
            descs = [
                pltpu.async_copy(ones_v, dacc.at[didx.at[j + k]], sem, add=True)
                for k in range(DEG_K)
            ]
            for desc in descs:
                desc.wait()

        plsc.subcore_barrier()
        pltpu.sync_copy(dacc.at[pl.ds(rbase, rpt)],
                        out_hbm.at[cid].at[pl.ds(rbase, rpt)])

    return deg_kernel(dst2, zeros_nd, ones2d)


def _aggregate_call(hs, src2, dst2, np_, d, n_chunks):
    @functools.partial(
        pl.kernel,
        out_type=jax.ShapeDtypeStruct((NC, np_, d), jnp.float32),
        mesh=plsc.VectorSubcoreMesh(core_axis_name="c", subcore_axis_name="s",
                                    num_cores=NC, num_subcores=NS),
        scratch_types=[
            pltpu.VMEM((BSZ, CH), jnp.int32),
            pltpu.VMEM((BSZ, CH), jnp.int32),
            pltpu.VMEM((CH, d), jnp.float32),
            pltpu.VMEM((CH, d), jnp.float32),
            pltpu.VMEM_SHARED((np_, d), jnp.float32),
            pltpu.SemaphoreType.DMA,
            pltpu.SemaphoreType.DMA,
        ],
    )
    def agg_kernel(hs_hbm, src_hbm, dst_hbm, out_hbm,
                   sidx, didx, rows_a, rows_b, acc, sem_a, sem_b):
        cid = lax.axis_index("c")
        sid = lax.axis_index("s")
        rpt = np_ // NS
        rbase = sid * rpt
        pltpu.sync_copy(hs_hbm.at[pl.ds(rbase, rpt)], acc.at[pl.ds(rbase, rpt)])
        plsc.subcore_barrier()

        def process(base_chunk, nblocks):
            for blk in range(nblocks):
                cb = base_chunk + blk * BSZ
                pltpu.sync_copy(src_hbm.at[pl.ds(cb, BSZ)], sidx)
                pltpu.sync_copy(dst_hbm.at[pl.ds(cb, BSZ)], didx)
                pltpu.async_copy(hs_hbm.at[sidx.at[0]], rows_a, sem_a).wait()

                @pl.loop(0, BSZ, step=2)
                def _(j):
                    gb = pltpu.async_copy(hs_hbm.at[sidx.at[j + 1]], rows_b,
                                          sem_b)
                    pltpu.sync_copy(rows_a, acc.at[didx.at[j]], add=True)
                    gb.wait()

                    @pl.when(j + 2 < BSZ)
                    def _():
                        pltpu.async_copy(hs_hbm.at[sidx.at[j + 2]], rows_a,
                                         sem_a)

                    pltpu.sync_copy(rows_b, acc.at[didx.at[j + 1]], add=True)

                    @pl.when(j + 2 < BSZ)
                    def _():
                        pltpu.make_async_copy(hs_hbm.at[sidx.at[0]], rows_a,
                                              sem_a).wait()

        k0 = (n_chunks * 2 * CORE0_NUM) // CORE_DEN
        k1 = n_chunks * 2 - k0

        @pl.when(cid == 0)
        def _():
            process(sid * k0, k0 // BSZ)

        @pl.when(cid == 1)
        def _():
            process(NS * k0 + sid * k1, k1 // BSZ)

        plsc.subcore_barrier()
        pltpu.sync_copy(acc.at[pl.ds(rbase, rpt)],
                        out_hbm.at[cid].at[pl.ds(rbase, rpt)])

    return agg_kernel(hs, src2, dst2)


def _matmul_call(x, w):
    def mm_kernel(x_ref, w_ref, o_ref):
        o_ref[...] = jnp.dot(x_ref[...], w_ref[...],
                             preferred_element_type=jnp.float32)

    return pl.pallas_call(
        mm_kernel,
        out_shape=jax.ShapeDtypeStruct((x.shape[0], w.shape[1]), jnp.float32),
    )(x, w)


def _scale_first_call(h, degs, np_, d):
    def k(h_ref, dg_ref, hs_ref, dinv_ref):
        deg = dg_ref[0, :, 0:1] + dg_ref[1, :, 0:1] + 1.0
        dinv = lax.rsqrt(deg)
        dinv_ref[...] = dinv
        hs_ref[...] = h_ref[...] * dinv

    return pl.pallas_call(
        k,
        out_shape=[
            jax.ShapeDtypeStruct((np_, d), jnp.float32),
            jax.ShapeDtypeStruct((np_, 1), jnp.float32),
        ],
    )(h, degs)


def _mid_layer_call(acc, hs_prev, dinv, b, g, be, w_next, n, np_, d):
    def k(acc_ref, hsp_ref, dinv_ref, b_ref, g_ref, be_ref, w_ref, o_ref):
        dinv_v = dinv_ref[...]
        agg = acc_ref[0] + acc_ref[1] - hsp_ref[...]
        y = agg * dinv_v + b_ref[...]
        yr = y[:n, :]
        mu = jnp.mean(yr, axis=0, keepdims=True)
        cr = yr - mu
        var = jnp.mean(cr * cr, axis=0, keepdims=True)
        yn = g_ref[...] * ((y - mu) * lax.rsqrt(var + EPS)) + be_ref[...]
        r = jnp.maximum(yn, 0.0)
        o_ref[...] = jnp.dot(r, w_ref[...],
                             preferred_element_type=jnp.float32) * dinv_v

    return pl.pallas_call(
        k,
        out_shape=jax.ShapeDtypeStruct((np_, d), jnp.float32),
    )(acc, hs_prev, dinv, b, g, be, w_next)


def _final_layer_call(acc, hs_prev, dinv, b, n, d):
    def k(acc_ref, hsp_ref, dinv_ref, b_ref, o_ref):
        agg = acc_ref[0, :n, :] + acc_ref[1, :n, :] - hsp_ref[:n, :]
        o_ref[...] = agg * dinv_ref[:n, :] + b_ref[...]

    return pl.pallas_call(
        k,
        out_shape=jax.ShapeDtypeStruct((n, d), jnp.float32),
    )(acc, hs_prev, dinv, b)


def kernel(x, edge_index, W1, b1, g1, be1, W2, b2, g2, be2, W3, b3):
    n, d = x.shape
    e = edge_index.shape[1]

    np_ = (n // 128 + 1) * 128
    epw = -(-e // (NW * CH * 8)) * CH * 8
    ep = epw * NW
    n_chunks = epw // CH
    pad = ep - e
    src = edge_index[0]
    dst = edge_index[1]
    pad_rows = n + jnp.arange(pad, dtype=jnp.int32) % (np_ - n)
    src2 = jnp.concatenate([src, pad_rows]).reshape(-1, CH)
    dst2 = jnp.concatenate([dst, pad_rows]).reshape(-1, CH)
    zeros_nd = jnp.zeros((np_, d), jnp.float32)
    ones2d = jnp.ones((CH, d), jnp.float32)
    x_p = jnp.concatenate([x, jnp.zeros((np_ - n, d), jnp.float32)])

    degs = _degree_call(dst2, zeros_nd, ones2d, np_, d, n_chunks)
    h1 = _matmul_call(x_p, W1)
    hs1, dinv = _scale_first_call(h1, degs, np_, d)

    b1r, g1r, be1r = b1.reshape(1, d), g1.reshape(1, d), be1.reshape(1, d)
    b2r, g2r, be2r = b2.reshape(1, d), g2.reshape(1, d), be2.reshape(1, d)
    b3r = b3.reshape(1, d)

    acc1 = _aggregate_call(hs1, src2, dst2, np_, d, n_chunks)
    hs2 = _mid_layer_call(acc1, hs1, dinv, b1r, g1r, be1r, W2, n, np_, d)
    acc2 = _aggregate_call(hs2, src2, dst2, np_, d, n_chunks)
    hs3 = _mid_layer_call(acc2, hs2, dinv, b2r, g2r, be2r, W3, n, np_, d)
    acc3 = _aggregate_call(hs3, src2, dst2, np_, d, n_chunks)
    return _final_layer_call(acc3, hs3, dinv, b3r, n, d)


# speedup vs baseline: 3.0828x; 1.1372x over previous
import dataclasses
import functools

import jax
import jax.numpy as jnp
from jax import lax
from jax.experimental import pallas as pl
from jax.experimental.pallas import tpu as pltpu
from jax.experimental.pallas import tpu_sc as plsc

NC = 2
NS = 16
NW = NC * NS
CH = 128
DEG_K = 8
BSZ = 16
CORE0_NUM, CORE_DEN = 1, 2
EPS = 1e-5


_SC_COMPILER_PARAMS = pltpu.CompilerParams()
if "needs_layout_passes" in pltpu.CompilerParams.__dataclass_fields__:
    _SC_COMPILER_PARAMS = dataclasses.replace(
        _SC_COMPILER_PARAMS, needs_layout_passes=False)


def _degree_call(dst2, np_, n_chunks):
    @functools.partial(
        pl.kernel,
        out_type=jax.ShapeDtypeStruct((NW, np_), jnp.float32),
        mesh=plsc.VectorSubcoreMesh(core_axis_name="c", subcore_axis_name="s",
                                    num_cores=NC, num_subcores=NS),
        compiler_params=_SC_COMPILER_PARAMS,
        scratch_types=[
            pltpu.VMEM((n_chunks, CH), jnp.int32),
            pltpu.VMEM((np_,), jnp.float32),
        ],
    )
    def deg_kernel(dst_hbm, out_hbm, didx, hist):
        cid = lax.axis_index("c")
        sid = lax.axis_index("s")
        wid = sid * NC + cid

        @pl.loop(0, np_, step=16)
        def _(i):
            hist[pl.ds(i, 16)] = jnp.zeros((16,), jnp.float32)

        pltpu.sync_copy(dst_hbm.at[pl.ds(wid * n_chunks, n_chunks)], didx)
        ones = jnp.ones((16,), jnp.float32)

        @pl.loop(0, n_chunks)
        def _(c):
            @pl.loop(0, CH, step=16)
            def _(i):
                plsc.addupdate_scatter(hist, [didx[c, pl.ds(i, 16)]], ones)

        pltpu.sync_copy(hist, out_hbm.at[wid])

    return deg_kernel(dst2)


def _aggregate_call(hs, src2, dst2, np_, d, n_chunks):
    @functools.partial(
        pl.kernel,
        out_type=jax.ShapeDtypeStruct((NC, np_, d), jnp.float32),
        mesh=plsc.VectorSubcoreMesh(core_axis_name="c", subcore_axis_name="s",
                                    num_cores=NC, num_subcores=NS),
        scratch_types=[
            pltpu.VMEM((BSZ, CH), jnp.int32),
            pltpu.VMEM((BSZ, CH), jnp.int32),
            pltpu.VMEM((CH, d), jnp.float32),
            pltpu.VMEM((CH, d), jnp.float32),
            pltpu.VMEM_SHARED((np_, d), jnp.float32),
            pltpu.SemaphoreType.DMA,
            pltpu.SemaphoreType.DMA,
        ],
    )
    def agg_kernel(hs_hbm, src_hbm, dst_hbm, out_hbm,
                   sidx, didx, rows_a, rows_b, acc, sem_a, sem_b):
        cid = lax.axis_index("c")
        sid = lax.axis_index("s")
        rpt = np_ // NS
        rbase = sid * rpt
        pltpu.sync_copy(hs_hbm.at[pl.ds(rbase, rpt)], acc.at[pl.ds(rbase, rpt)])
        plsc.subcore_barrier()

        def process(base_chunk, nblocks):
            for blk in range(nblocks):
                cb = base_chunk + blk * BSZ
                pltpu.sync_copy(src_hbm.at[pl.ds(cb, BSZ)], sidx)
                pltpu.sync_copy(dst_hbm.at[pl.ds(cb, BSZ)], didx)
                pltpu.async_copy(hs_hbm.at[sidx.at[0]], rows_a, sem_a).wait()

                @pl.loop(0, BSZ, step=2)
                def _(j):
                    gb = pltpu.async_copy(hs_hbm.at[sidx.at[j + 1]], rows_b,
                                          sem_b)
                    pltpu.sync_copy(rows_a, acc.at[didx.at[j]], add=True)
                    gb.wait()

                    @pl.when(j + 2 < BSZ)
                    def _():
                        pltpu.async_copy(hs_hbm.at[sidx.at[j + 2]], rows_a,
                                         sem_a)

                    pltpu.sync_copy(rows_b, acc.at[didx.at[j + 1]], add=True)

                    @pl.when(j + 2 < BSZ)
                    def _():
                        pltpu.make_async_copy(hs_hbm.at[sidx.at[0]], rows_a,
                                              sem_a).wait()

        k0 = (n_chunks * 2 * CORE0_NUM) // CORE_DEN
        k1 = n_chunks * 2 - k0

        @pl.when(cid == 0)
        def _():
            process(sid * k0, k0 // BSZ)

        @pl.when(cid == 1)
        def _():
            process(NS * k0 + sid * k1, k1 // BSZ)

        plsc.subcore_barrier()
        pltpu.sync_copy(acc.at[pl.ds(rbase, rpt)],
                        out_hbm.at[cid].at[pl.ds(rbase, rpt)])

    return agg_kernel(hs, src2, dst2)


def _matmul_call(x, w):
    def mm_kernel(x_ref, w_ref, o_ref):
        o_ref[...] = jnp.dot(x_ref[...], w_ref[...],
                             preferred_element_type=jnp.float32)

    return pl.pallas_call(
        mm_kernel,
        out_shape=jax.ShapeDtypeStruct((x.shape[0], w.shape[1]), jnp.float32),
    )(x, w)


def _scale_first_call(h, degs, np_, d):
    def k(h_ref, dg_ref, hs_ref, dinv_ref):
        deg = jnp.sum(dg_ref[...], axis=0, keepdims=True) + 1.0
        dinv = jnp.transpose(lax.rsqrt(deg))
        dinv_ref[...] = dinv
        hs_ref[...] = h_ref[...] * dinv

    return pl.pallas_call(
        k,
        out_shape=[
            jax.ShapeDtypeStruct((np_, d), jnp.float32),
            jax.ShapeDtypeStruct((np_, 1), jnp.float32),
        ],
    )(h, degs)


def _mid_layer_call(acc, hs_prev, dinv, b, g, be, w_next, n, np_, d):
    def k(acc_ref, hsp_ref, dinv_ref, b_ref, g_ref, be_ref, w_ref, o_ref):
        dinv_v = dinv_ref[...]
        agg = acc_ref[0] + acc_ref[1] - hsp_ref[...]
        y = agg * dinv_v + b_ref[...]
        yr = y[:n, :]
        mu = jnp.mean(yr, axis=0, keepdims=True)
        cr = yr - mu
        var = jnp.mean(cr * cr, axis=0, keepdims=True)
        yn = g_ref[...] * ((y - mu) * lax.rsqrt(var + EPS)) + be_ref[...]
        r = jnp.maximum(yn, 0.0)
        o_ref[...] = jnp.dot(r, w_ref[...],
                             preferred_element_type=jnp.float32) * dinv_v

    return pl.pallas_call(
        k,
        out_shape=jax.ShapeDtypeStruct((np_, d), jnp.float32),
    )(acc, hs_prev, dinv, b, g, be, w_next)


def _final_layer_call(acc, hs_prev, dinv, b, n, d):
    def k(acc_ref, hsp_ref, dinv_ref, b_ref, o_ref):
        agg = acc_ref[0, :n, :] + acc_ref[1, :n, :] - hsp_ref[:n, :]
        o_ref[...] = agg * dinv_ref[:n, :] + b_ref[...]

    return pl.pallas_call(
        k,
        out_shape=jax.ShapeDtypeStruct((n, d), jnp.float32),
    )(acc, hs_prev, dinv, b)


def kernel(x, edge_index, W1, b1, g1, be1, W2, b2, g2, be2, W3, b3):
    n, d = x.shape
    e = edge_index.shape[1]

    np_ = (n // 128 + 1) * 128
    epw = -(-e // (NW * CH * 8)) * CH * 8
    ep = epw * NW
    n_chunks = epw // CH
    pad = ep - e
    src = edge_index[0]
    dst = edge_index[1]
    pad_rows = n + jnp.arange(pad, dtype=jnp.int32) % (np_ - n)
    src2 = jnp.concatenate([src, pad_rows]).reshape(-1, CH)
    dst2 = jnp.concatenate([dst, pad_rows]).reshape(-1, CH)
    x_p = jnp.concatenate([x, jnp.zeros((np_ - n, d), jnp.float32)])

    degs = _degree_call(dst2, np_, n_chunks)
    h1 = _matmul_call(x_p, W1)
    hs1, dinv = _scale_first_call(h1, degs, np_, d)

    b1r, g1r, be1r = b1.reshape(1, d), g1.reshape(1, d), be1.reshape(1, d)
    b2r, g2r, be2r = b2.reshape(1, d), g2.reshape(1, d), be2.reshape(1, d)
    b3r = b3.reshape(1, d)

    acc1 = _aggregate_call(hs1, src2, dst2, np_, d, n_chunks)
    hs2 = _mid_layer_call(acc1, hs1, dinv, b1r, g1r, be1r, W2, n, np_, d)
    acc2 = _aggregate_call(hs2, src2, dst2, np_, d, n_chunks)
    hs3 = _mid_layer_call(acc2, hs2, dinv, b2r, g2r, be2r, W3, n, np_, d)
    acc3 = _aggregate_call(hs3, src2, dst2, np_, d, n_chunks)
    return _final_layer_call(acc3, hs3, dinv, b3r, n, d)
